# Initial kernel scaffold; baseline (speedup 1.0000x reference)
import jax
import jax.numpy as jnp
from jax.experimental import pallas as pl


def kernel(p1_key_points, p2_key_points, p2_key_points_sym):
    raise NotImplementedError("write your pallas kernel here")


# speedup vs baseline: 1.5203x; 1.5203x over previous
import functools

import jax
import jax.numpy as jnp
from jax import lax
from jax.experimental import pallas as pl
from jax.experimental.pallas import tpu as pltpu
from jax.experimental.pallas import tpu_sc as plsc

Q = 16384
K = 3600
D = 12
DP = 16
BQ = 512


def _knn_body(p1_ref, bt1_ref, bt2_ref, out_ref):
    a = p1_ref[...]
    an = jnp.sum(a * a, axis=1, keepdims=True)

    def min_arg(bt_ref):
        b = bt_ref[...]
        bn = jnp.sum(b * b, axis=0, keepdims=True)
        ab = lax.dot_general(a, b, (((1,), (0,)), ((), ())),
                             preferred_element_type=jnp.float32)
        d = an + bn - 2.0 * ab
        dmin = jnp.min(d, axis=1, keepdims=True)
        col = lax.broadcasted_iota(jnp.int32, d.shape, 1)
        imin = jnp.min(jnp.where(d == dmin, col, K),
                       axis=1, keepdims=True)
        return dmin, imin

    dmin, imin = min_arg(bt1_ref)
    dsmin, ismin = min_arg(bt2_ref)
    out_ref[...] = jnp.where(dmin < dsmin, imin, ismin + K)


def _knn_indices(p1, bt1, bt2):
    grid = (Q // BQ,)
    return pl.pallas_call(
        _knn_body,
        grid=grid,
        in_specs=[
            pl.BlockSpec((BQ, D), lambda i: (i, 0)),
            pl.BlockSpec((D, K), lambda i: (0, 0)),
            pl.BlockSpec((D, K), lambda i: (0, 0)),
        ],
        out_specs=pl.BlockSpec((BQ, 1), lambda i: (i, 0)),
        out_shape=jax.ShapeDtypeStruct((Q, 1), jnp.int32),
    )(p1, bt1, bt2)


_SC_INFO = plsc.get_sparse_core_info()
_NC = _SC_INFO.num_cores
_NS = _SC_INFO.num_subcores
_NW = _NC * _NS
_BPW = Q // _NW


@functools.partial(
    pl.kernel,
    mesh=plsc.VectorSubcoreMesh(core_axis_name="c", subcore_axis_name="s"),
    out_type=jax.ShapeDtypeStruct((Q, DP), jnp.float32),
    scratch_types=[
        pltpu.VMEM((_BPW,), jnp.int32),
        pltpu.VMEM((_BPW, DP), jnp.float32),
        pltpu.SemaphoreType.DMA,
    ],
    compiler_params=pltpu.CompilerParams(use_tc_tiling_on_sc=False),
)
def _sc_gather(table_hbm, idx_hbm, out_hbm, idx_v, rows_v, sem):
    wid = lax.axis_index("s") * _NC + lax.axis_index("c")
    base = wid * _BPW
    pltpu.sync_copy(idx_hbm.at[pl.ds(base, _BPW)], idx_v)
    pltpu.async_copy(table_hbm.at[idx_v], rows_v, sem).wait()
    pltpu.sync_copy(rows_v, out_hbm.at[pl.ds(base, _BPW)])


def kernel(p1_key_points, p2_key_points, p2_key_points_sym):
    bt1 = p2_key_points.T
    bt2 = p2_key_points_sym.T
    inds = _knn_indices(p1_key_points, bt1, bt2)
    table = jnp.concatenate([p2_key_points, p2_key_points_sym], axis=0)
    table = jnp.pad(table, ((0, 0), (0, DP - D)))
    matched = _sc_gather(table, inds.reshape(Q))
    return matched[:, :D]
